# Initial kernel scaffold; baseline (speedup 1.0000x reference)
import jax
import jax.numpy as jnp
from jax.experimental import pallas as pl


def kernel(dense_xyz, sparse_xyz, dense_feature, sparse_feature, W1, g1, b1, W2, g2, b2):
    raise NotImplementedError("write your pallas kernel here")


# speedup vs baseline: 23.2170x; 23.2170x over previous
import functools
import jax
import jax.numpy as jnp
from jax.experimental import pallas as pl
from jax.experimental.pallas import tpu as pltpu


def _pass_a(dx_ref, sx_ref, df_ref, sf_ref, w1a_ref, w1b_ref,
            y1_ref, st_ref, z_ref, *, n2, eps):
    b = pl.program_id(0)
    j = pl.program_id(1)
    bf16 = jnp.bfloat16

    @pl.when(j == 0)
    def _():
        z_ref[...] = jnp.dot(w1a_ref[...].astype(bf16),
                             sf_ref[0].astype(bf16),
                             preferred_element_type=jnp.float32).astype(bf16)

    dx = dx_ref[0]
    sx = sx_ref[0]
    qk = jax.lax.dot_general(dx.astype(bf16), sx.astype(bf16),
                             (((0,), (0,)), ((), ())),
                             preferred_element_type=jnp.float32)
    kk = jnp.sum(sx * sx, axis=0, keepdims=True)
    sel = kk - 2.0 * qk
    dx2 = dx * dx
    qq_col = jax.lax.dot_general(
        dx2, jnp.ones((3, 8), jnp.float32), (((0,), (0,)), ((), ())),
        preferred_element_type=jnp.float32,
        precision=jax.lax.Precision.HIGHEST)[:, 0:1]

    iota = jax.lax.broadcasted_iota(jnp.int32, sel.shape, 1)
    d = sel
    mins = []
    idxs = []
    for t in range(3):
        m = jnp.min(d, axis=1, keepdims=True)
        i = jnp.min(jnp.where(d == m, iota, n2), axis=1, keepdims=True)
        mins.append(m)
        idxs.append(i)
        if t < 2:
            d = jnp.where(iota == i, jnp.float32(1e30), d)

    invs = [1.0 / jnp.maximum(jnp.maximum(m + qq_col, 0.0), eps) for m in mins]
    tot = invs[0] + invs[1] + invs[2]
    s_mat = (jnp.where(iota == idxs[0], invs[0] / tot, 0.0)
             + jnp.where(iota == idxs[1], invs[1] / tot, 0.0)
             + jnp.where(iota == idxs[2], invs[2] / tot, 0.0))

    y = (jnp.dot(w1b_ref[...].astype(bf16), df_ref[0].astype(bf16),
                 preferred_element_type=jnp.float32)
         + jax.lax.dot_general(z_ref[...], s_mat.astype(bf16),
                               (((1,), (1,)), ((), ())),
                               preferred_element_type=jnp.float32))
    y1_ref[...] = y[None]

    @pl.when((b == 0) & (j == 0))
    def _():
        st_ref[...] = jnp.zeros_like(st_ref)

    st_ref[...] += jnp.concatenate(
        [jnp.sum(y, axis=1, keepdims=True),
         jnp.sum(y * y, axis=1, keepdims=True)], axis=1)


def _pass_b(y1_ref, st_ref, g_ref, bb_ref, w2_ref, y2_ref, st2_ref, *, cnt):
    b = pl.program_id(0)
    j = pl.program_id(1)
    st = st_ref[...]
    m = st[:, 0:1] * (1.0 / cnt)
    v = st[:, 1:2] * (1.0 / cnt) - m * m
    a = g_ref[...] * jax.lax.rsqrt(v + 1e-5)
    sh = bb_ref[...] - m * a
    yh = jnp.maximum(y1_ref[0] * a + sh, 0.0)
    y2 = jnp.dot(w2_ref[...].astype(jnp.bfloat16), yh.astype(jnp.bfloat16),
                 preferred_element_type=jnp.float32)
    y2_ref[...] = y2[None]

    @pl.when((b == 0) & (j == 0))
    def _():
        st2_ref[...] = jnp.zeros_like(st2_ref)

    st2_ref[...] += jnp.concatenate(
        [jnp.sum(y2, axis=1, keepdims=True),
         jnp.sum(y2 * y2, axis=1, keepdims=True)], axis=1)


def _pass_c(y2_ref, st_ref, g_ref, bb_ref, out_ref, *, cnt):
    st = st_ref[...]
    m = st[:, 0:1] * (1.0 / cnt)
    v = st[:, 1:2] * (1.0 / cnt) - m * m
    a = g_ref[...] * jax.lax.rsqrt(v + 1e-5)
    sh = bb_ref[...] - m * a
    out_ref[...] = jnp.maximum(y2_ref[...] * a + sh, 0.0)


def kernel(dense_xyz, sparse_xyz, dense_feature, sparse_feature,
           W1, g1, b1, W2, g2, b2):
    B, _, N1 = dense_xyz.shape
    N2 = sparse_xyz.shape[2]
    C1 = dense_feature.shape[1]
    C2 = sparse_feature.shape[1]
    C = W1.shape[0]
    C3 = W2.shape[0]
    f32 = jnp.float32

    W1a = W1[:, :C2]
    W1b = W1[:, C2:]
    g1c = g1.reshape(C, 1)
    b1c = b1.reshape(C, 1)
    g2c = g2.reshape(C3, 1)
    b2c = b2.reshape(C3, 1)

    nblk = min(512, N1)
    nj = N1 // nblk
    cnt = float(B * N1)

    y1, st1 = pl.pallas_call(
        functools.partial(_pass_a, n2=N2, eps=1e-10),
        grid=(B, nj),
        in_specs=[
            pl.BlockSpec((1, 3, nblk), lambda b, j: (b, 0, j)),
            pl.BlockSpec((1, 3, N2), lambda b, j: (b, 0, 0)),
            pl.BlockSpec((1, C1, nblk), lambda b, j: (b, 0, j)),
            pl.BlockSpec((1, C2, N2), lambda b, j: (b, 0, 0)),
            pl.BlockSpec((C, C2), lambda b, j: (0, 0)),
            pl.BlockSpec((C, C1), lambda b, j: (0, 0)),
        ],
        out_specs=[
            pl.BlockSpec((1, C, nblk), lambda b, j: (b, 0, j)),
            pl.BlockSpec((C, 2), lambda b, j: (0, 0)),
        ],
        out_shape=[
            jax.ShapeDtypeStruct((B, C, N1), f32),
            jax.ShapeDtypeStruct((C, 2), f32),
        ],
        scratch_shapes=[pltpu.VMEM((C, N2), jnp.bfloat16)],
    )(dense_xyz, sparse_xyz, dense_feature, sparse_feature, W1a, W1b)

    nblk2 = min(1024, N1)
    nj2 = N1 // nblk2
    y2, st2 = pl.pallas_call(
        functools.partial(_pass_b, cnt=cnt),
        grid=(B, nj2),
        in_specs=[
            pl.BlockSpec((1, C, nblk2), lambda b, j: (b, 0, j)),
            pl.BlockSpec((C, 2), lambda b, j: (0, 0)),
            pl.BlockSpec((C, 1), lambda b, j: (0, 0)),
            pl.BlockSpec((C, 1), lambda b, j: (0, 0)),
            pl.BlockSpec((C3, C), lambda b, j: (0, 0)),
        ],
        out_specs=[
            pl.BlockSpec((1, C3, nblk2), lambda b, j: (b, 0, j)),
            pl.BlockSpec((C3, 2), lambda b, j: (0, 0)),
        ],
        out_shape=[
            jax.ShapeDtypeStruct((B, C3, N1), f32),
            jax.ShapeDtypeStruct((C3, 2), f32),
        ],
    )(y1, st1, g1c, b1c, W2)

    out = pl.pallas_call(
        functools.partial(_pass_c, cnt=cnt),
        grid=(B, nj2),
        in_specs=[
            pl.BlockSpec((1, C3, nblk2), lambda b, j: (b, 0, j)),
            pl.BlockSpec((C3, 2), lambda b, j: (0, 0)),
            pl.BlockSpec((C3, 1), lambda b, j: (0, 0)),
            pl.BlockSpec((C3, 1), lambda b, j: (0, 0)),
        ],
        out_specs=pl.BlockSpec((1, C3, nblk2), lambda b, j: (b, 0, j)),
        out_shape=jax.ShapeDtypeStruct((B, C3, N1), f32),
    )(y2, st2, g2c, b2c)

    return out


# speedup vs baseline: 24.2852x; 1.0460x over previous
import functools
import jax
import jax.numpy as jnp
from jax.experimental import pallas as pl
from jax.experimental.pallas import tpu as pltpu


def _pass_a(dx_ref, sx_ref, df_ref, sf_ref, w1a_ref, w1b_ref,
            y1_ref, st_ref, z_ref, s_ref, *, n2, eps):
    b = pl.program_id(0)
    j = pl.program_id(1)
    bf16 = jnp.bfloat16

    @pl.when(j == 0)
    def _():
        z_ref[...] = jnp.dot(w1a_ref[...].astype(bf16),
                             sf_ref[0].astype(bf16),
                             preferred_element_type=jnp.float32).astype(bf16)

    dx = dx_ref[0]
    sx = sx_ref[0]
    qk = jax.lax.dot_general(dx.astype(bf16), sx.astype(bf16),
                             (((0,), (0,)), ((), ())),
                             preferred_element_type=jnp.float32)
    kk = jnp.sum(sx * sx, axis=0, keepdims=True)
    sel = 0.5 * kk - qk
    dx2 = dx * dx
    qq_col = jax.lax.dot_general(
        dx2, jnp.ones((3, 8), jnp.float32), (((0,), (0,)), ((), ())),
        preferred_element_type=jnp.float32,
        precision=jax.lax.Precision.HIGHEST)[:, 0:1]

    big = jnp.float32(1e30)
    nblk = sel.shape[0]

    def weights(m1, m2, m3):
        invs = [1.0 / jnp.maximum(jnp.maximum(2.0 * m + qq_col, 0.0), eps)
                for m in (m1, m2, m3)]
        tot = invs[0] + invs[1] + invs[2]
        return invs[0] / tot, invs[1] / tot, invs[2] / tot

    m1 = jnp.min(sel, axis=1, keepdims=True)
    eq1 = sel == m1
    d1 = jnp.where(eq1, big, sel)
    m2 = jnp.min(d1, axis=1, keepdims=True)
    eq2 = d1 == m2
    d2 = jnp.where(eq2, big, d1)
    m3 = jnp.min(d2, axis=1, keepdims=True)
    eq3 = d2 == m3
    w1, w2, w3 = weights(m1, m2, m3)
    s_mat = (jnp.where(eq1, w1, 0.0) + jnp.where(eq2, w2, 0.0)
             + jnp.where(eq3, w3, 0.0))
    s_ref[...] = s_mat.astype(bf16)
    cnt = jnp.sum(s_mat)

    @pl.when(jnp.abs(cnt - nblk) > 1e-3)
    def _():
        iota = jax.lax.broadcasted_iota(
            jnp.int32, sel.shape, 1).astype(jnp.float32)
        d = sel
        mins = []
        idxs = []
        for t in range(3):
            m = jnp.min(d, axis=1, keepdims=True)
            i = jnp.min(jnp.where(d == m, iota, jnp.float32(n2)),
                        axis=1, keepdims=True)
            mins.append(m)
            idxs.append(i)
            if t < 2:
                d = jnp.where(iota == i, big, d)
        v1, v2, v3 = weights(*mins)
        s_ex = (jnp.where(iota == idxs[0], v1, 0.0)
                + jnp.where(iota == idxs[1], v2, 0.0)
                + jnp.where(iota == idxs[2], v3, 0.0))
        s_ref[...] = s_ex.astype(bf16)

    y = (jnp.dot(w1b_ref[...].astype(bf16), df_ref[0].astype(bf16),
                 preferred_element_type=jnp.float32)
         + jax.lax.dot_general(z_ref[...], s_ref[...],
                               (((1,), (1,)), ((), ())),
                               preferred_element_type=jnp.float32))
    y1_ref[...] = y.astype(bf16)[None]

    @pl.when((b == 0) & (j == 0))
    def _():
        st_ref[...] = jnp.zeros_like(st_ref)

    st_ref[...] += jnp.concatenate(
        [jnp.sum(y, axis=1, keepdims=True),
         jnp.sum(y * y, axis=1, keepdims=True)], axis=1)


def _pass_b(y1_ref, st_ref, g_ref, bb_ref, w2_ref, y2_ref, st2_ref, *, cnt):
    b = pl.program_id(0)
    j = pl.program_id(1)
    st = st_ref[...]
    m = st[:, 0:1] * (1.0 / cnt)
    v = st[:, 1:2] * (1.0 / cnt) - m * m
    a = g_ref[...] * jax.lax.rsqrt(v + 1e-5)
    sh = bb_ref[...] - m * a
    yh = jnp.maximum(y1_ref[0].astype(jnp.float32) * a + sh, 0.0)
    y2 = jnp.dot(w2_ref[...].astype(jnp.bfloat16), yh.astype(jnp.bfloat16),
                 preferred_element_type=jnp.float32)
    y2_ref[...] = y2.astype(jnp.bfloat16)[None]

    @pl.when((b == 0) & (j == 0))
    def _():
        st2_ref[...] = jnp.zeros_like(st2_ref)

    st2_ref[...] += jnp.concatenate(
        [jnp.sum(y2, axis=1, keepdims=True),
         jnp.sum(y2 * y2, axis=1, keepdims=True)], axis=1)


def _pass_c(y2_ref, st_ref, g_ref, bb_ref, out_ref, *, cnt):
    st = st_ref[...]
    m = st[:, 0:1] * (1.0 / cnt)
    v = st[:, 1:2] * (1.0 / cnt) - m * m
    a = g_ref[...] * jax.lax.rsqrt(v + 1e-5)
    sh = bb_ref[...] - m * a
    out_ref[...] = jnp.maximum(y2_ref[...].astype(jnp.float32) * a + sh, 0.0)


def kernel(dense_xyz, sparse_xyz, dense_feature, sparse_feature,
           W1, g1, b1, W2, g2, b2):
    B, _, N1 = dense_xyz.shape
    N2 = sparse_xyz.shape[2]
    C1 = dense_feature.shape[1]
    C2 = sparse_feature.shape[1]
    C = W1.shape[0]
    C3 = W2.shape[0]
    f32 = jnp.float32

    W1a = W1[:, :C2]
    W1b = W1[:, C2:]
    g1c = g1.reshape(C, 1)
    b1c = b1.reshape(C, 1)
    g2c = g2.reshape(C3, 1)
    b2c = b2.reshape(C3, 1)

    nblk = min(512, N1)
    nj = N1 // nblk
    cnt = float(B * N1)

    y1, st1 = pl.pallas_call(
        functools.partial(_pass_a, n2=N2, eps=1e-10),
        grid=(B, nj),
        in_specs=[
            pl.BlockSpec((1, 3, nblk), lambda b, j: (b, 0, j)),
            pl.BlockSpec((1, 3, N2), lambda b, j: (b, 0, 0)),
            pl.BlockSpec((1, C1, nblk), lambda b, j: (b, 0, j)),
            pl.BlockSpec((1, C2, N2), lambda b, j: (b, 0, 0)),
            pl.BlockSpec((C, C2), lambda b, j: (0, 0)),
            pl.BlockSpec((C, C1), lambda b, j: (0, 0)),
        ],
        out_specs=[
            pl.BlockSpec((1, C, nblk), lambda b, j: (b, 0, j)),
            pl.BlockSpec((C, 2), lambda b, j: (0, 0)),
        ],
        out_shape=[
            jax.ShapeDtypeStruct((B, C, N1), jnp.bfloat16),
            jax.ShapeDtypeStruct((C, 2), f32),
        ],
        scratch_shapes=[pltpu.VMEM((C, N2), jnp.bfloat16),
                        pltpu.VMEM((nblk, N2), jnp.bfloat16)],
    )(dense_xyz, sparse_xyz, dense_feature, sparse_feature, W1a, W1b)

    nblk2 = min(1024, N1)
    nj2 = N1 // nblk2
    y2, st2 = pl.pallas_call(
        functools.partial(_pass_b, cnt=cnt),
        grid=(B, nj2),
        in_specs=[
            pl.BlockSpec((1, C, nblk2), lambda b, j: (b, 0, j)),
            pl.BlockSpec((C, 2), lambda b, j: (0, 0)),
            pl.BlockSpec((C, 1), lambda b, j: (0, 0)),
            pl.BlockSpec((C, 1), lambda b, j: (0, 0)),
            pl.BlockSpec((C3, C), lambda b, j: (0, 0)),
        ],
        out_specs=[
            pl.BlockSpec((1, C3, nblk2), lambda b, j: (b, 0, j)),
            pl.BlockSpec((C3, 2), lambda b, j: (0, 0)),
        ],
        out_shape=[
            jax.ShapeDtypeStruct((B, C3, N1), jnp.bfloat16),
            jax.ShapeDtypeStruct((C3, 2), f32),
        ],
    )(y1, st1, g1c, b1c, W2)

    out = pl.pallas_call(
        functools.partial(_pass_c, cnt=cnt),
        grid=(B, nj2),
        in_specs=[
            pl.BlockSpec((1, C3, nblk2), lambda b, j: (b, 0, j)),
            pl.BlockSpec((C3, 2), lambda b, j: (0, 0)),
            pl.BlockSpec((C3, 1), lambda b, j: (0, 0)),
            pl.BlockSpec((C3, 1), lambda b, j: (0, 0)),
        ],
        out_specs=pl.BlockSpec((1, C3, nblk2), lambda b, j: (b, 0, j)),
        out_shape=jax.ShapeDtypeStruct((B, C3, N1), f32),
    )(y2, st2, g2c, b2c)

    return out
